# baseline (speedup 1.0000x reference)
import jax, jax.numpy as jnp
from jax import lax
from jax.experimental import pallas as pl

_ND = 50000; _NT = 50000; _B = 512

def _bdot(a, b):
    return lax.dot_general(a.astype(jnp.bfloat16), b.astype(jnp.bfloat16),
                           (((1,), (0,)), ((), ())),
                           preferred_element_type=jnp.float32)

def _gcn(x, ei, W, b, n):
    loop = jnp.arange(n, dtype=ei.dtype)
    src = jnp.concatenate([ei[0], loop])[::-1]
    dst = jnp.concatenate([ei[1], loop])[::-1]
    deg = jnp.zeros((n,), x.dtype).at[dst].add(1.0)
    dinv = 1.0 / jnp.sqrt(deg)
    norm = dinv[src] * dinv[dst]
    h = _bdot(x, W)
    out = jnp.zeros((n, W.shape[1]), x.dtype).at[dst].add(h[src] * norm[:, None])
    return out + b

def _gep(x, batch, nb):
    s = jax.ops.segment_sum(x, batch, num_segments=nb)
    c = jax.ops.segment_sum(jnp.ones((x.shape[0],), x.dtype), batch, num_segments=nb)
    return s / jnp.maximum(c, 1.0)[:, None]

def _ident(x):
    return pl.pallas_call(
        lambda x_ref, o_ref: o_ref.__setitem__((...,), x_ref[...]),
        out_shape=jax.ShapeDtypeStruct(x.shape, x.dtype),
    )(x)

def kernel(xd, xd_edge_index, xd_batch, xt, xt_edge_index, xt_batch, y,
           Wm1, bm1, Wm2, bm2, Wm3, bm3, Wmf1, bmf1, Wmf2, bmf2,
           Wp1, bp1, Wp2, bp2, Wp3, bp3, Wpf1, bpf1, Wpf2, bpf2,
           Wc1, bc1, Wc2, bc2, Wc3, bc3):
    relu = jax.nn.relu
    xd = relu(_gcn(xd, xd_edge_index, Wm1, bm1, _ND))
    xd = relu(_gcn(xd, xd_edge_index, Wm2, bm2, _ND))
    xd = relu(_gcn(xd, xd_edge_index, Wm3, bm3, _ND))
    xt = relu(_gcn(xt, xt_edge_index, Wp1, bp1, _NT))
    xt = relu(_gcn(xt, xt_edge_index, Wp2, bp2, _NT))
    xt = relu(_gcn(xt, xt_edge_index, Wp3, bp3, _NT))
    xdg = _gep(xd, xd_batch, _B)
    xdg = relu(_bdot(xdg, Wmf1) + bmf1)
    xdg = _bdot(xdg, Wmf2) + bmf2
    xtg = _gep(xt, xt_batch, _B)
    xtg = relu(_bdot(xtg, Wpf1) + bpf1)
    xtg = _bdot(xtg, Wpf2) + bpf2
    xj = jnp.concatenate([xdg, xtg], axis=1)
    h = relu(_bdot(xj, Wc1) + bc1)
    h = relu(_bdot(h, Wc2) + bc2)
    out = (_bdot(h, Wc3) + bc3).squeeze(1)
    return (_ident(out), y)


# speedup vs baseline: 1.0692x; 1.0692x over previous
import jax
import jax.numpy as jnp
from jax import lax
from jax.experimental import pallas as pl

_ND = 50000
_NT = 50000
_B = 512


def _bdot(a, b):
    return lax.dot_general(a.astype(jnp.bfloat16), b.astype(jnp.bfloat16),
                           (((1,), (0,)), ((), ())),
                           preferred_element_type=jnp.float32)


def _pmm(x, w):
    m, k = x.shape
    nout = w.shape[1]
    blk = 1000
    assert m % blk == 0

    def body(x_ref, w_ref, o_ref):
        o_ref[...] = _bdot(x_ref[...], w_ref[...])

    return pl.pallas_call(
        body,
        grid=(m // blk,),
        in_specs=[pl.BlockSpec((blk, k), lambda i: (i, 0)),
                  pl.BlockSpec((k, nout), lambda i: (0, 0))],
        out_specs=pl.BlockSpec((blk, nout), lambda i: (i, 0)),
        out_shape=jax.ShapeDtypeStruct((m, nout), jnp.float32),
    )(x, w)


def _branch(x, ei, Ws, bs, n):
    loop = jnp.arange(n, dtype=ei.dtype)
    src_all = jnp.concatenate([ei[0], loop])
    dst_all = jnp.concatenate([ei[1], loop])
    perm = jnp.argsort(dst_all, stable=True)
    src_s = src_all[perm]
    dst_s = dst_all[perm]
    deg = jnp.zeros((n,), jnp.float32).at[dst_s].add(
        1.0, indices_are_sorted=True)
    dinv = 1.0 / jnp.sqrt(deg)
    norm_s = dinv[src_s] * dinv[dst_s]
    h = x
    for W, b in zip(Ws, bs):
        hw = _pmm(h, W)
        upd = hw[src_s] * norm_s[:, None]
        out = jnp.zeros((n, W.shape[1]), jnp.float32).at[dst_s].add(
            upd, indices_are_sorted=True)
        h = jax.nn.relu(out + b)
    return h


def _gep(x, batch, nb):
    s = jnp.zeros((nb, x.shape[1]), x.dtype).at[batch].add(
        x, indices_are_sorted=True)
    c = jnp.zeros((nb,), x.dtype).at[batch].add(
        1.0, indices_are_sorted=True)
    return s / jnp.maximum(c, 1.0)[:, None]


def _head_body(xdg_ref, xtg_ref, Wmf1_ref, bmf1_ref, Wmf2_ref, bmf2_ref,
               Wpf1_ref, bpf1_ref, Wpf2_ref, bpf2_ref,
               Wc1_ref, bc1_ref, Wc2_ref, bc2_ref, Wc3_ref, bc3_ref, out_ref):
    relu = lambda v: jnp.maximum(v, 0.0)
    xdg = relu(_bdot(xdg_ref[...], Wmf1_ref[...]) + bmf1_ref[...])
    xdg = _bdot(xdg, Wmf2_ref[...]) + bmf2_ref[...]
    xtg = relu(_bdot(xtg_ref[...], Wpf1_ref[...]) + bpf1_ref[...])
    xtg = _bdot(xtg, Wpf2_ref[...]) + bpf2_ref[...]
    xj = jnp.concatenate([xdg, xtg], axis=1)
    h = relu(_bdot(xj, Wc1_ref[...]) + bc1_ref[...])
    h = relu(_bdot(h, Wc2_ref[...]) + bc2_ref[...])
    out_ref[...] = _bdot(h, Wc3_ref[...]) + bc3_ref[...]


def _head(xdg, xtg, Wmf1, bmf1, Wmf2, bmf2, Wpf1, bpf1, Wpf2, bpf2,
          Wc1, bc1, Wc2, bc2, Wc3, bc3):
    return pl.pallas_call(
        _head_body,
        out_shape=jax.ShapeDtypeStruct((_B, 1), jnp.float32),
    )(xdg, xtg, Wmf1, bmf1[None, :], Wmf2, bmf2[None, :],
      Wpf1, bpf1[None, :], Wpf2, bpf2[None, :],
      Wc1, bc1[None, :], Wc2, bc2[None, :], Wc3, bc3[None, :])


def kernel(xd, xd_edge_index, xd_batch, xt, xt_edge_index, xt_batch, y,
           Wm1, bm1, Wm2, bm2, Wm3, bm3, Wmf1, bmf1, Wmf2, bmf2,
           Wp1, bp1, Wp2, bp2, Wp3, bp3, Wpf1, bpf1, Wpf2, bpf2,
           Wc1, bc1, Wc2, bc2, Wc3, bc3):
    hd = _branch(xd, xd_edge_index, (Wm1, Wm2, Wm3), (bm1, bm2, bm3), _ND)
    ht = _branch(xt, xt_edge_index, (Wp1, Wp2, Wp3), (bp1, bp2, bp3), _NT)
    xdg = _gep(hd, xd_batch, _B)
    xtg = _gep(ht, xt_batch, _B)
    out = _head(xdg, xtg, Wmf1, bmf1, Wmf2, bmf2, Wpf1, bpf1, Wpf2, bpf2,
                Wc1, bc1, Wc2, bc2, Wc3, bc3)
    return (out[:, 0], y)
